# baseline (speedup 1.0000x reference)
import functools

import jax
import jax.numpy as jnp
from jax import lax
from jax.experimental import pallas as pl
from jax.experimental.pallas import tpu as pltpu
from jax.experimental.pallas import tpu_sc as plsc

NC = 2
NS = 16
K = 88
NI = 8
NR = 4
NU = 8
FRAC0 = 0.72


def _sc_scatter_call(table, idxpair, zeros, tpw0, tpw1):
    n, h = zeros.shape
    rpt = (n // NS) & ~7
    tail = n - NS * rpt
    assert tpw0 % NU == 0 and tpw1 % NU == 0
    mesh = plsc.VectorSubcoreMesh(core_axis_name="c", subcore_axis_name="s")

    @functools.partial(
        pl.kernel,
        mesh=mesh,
        out_type=jax.ShapeDtypeStruct((NC, n, h), jnp.float32),
        scratch_types=[
            pltpu.VMEM_SHARED((n, h), jnp.float32),
        ] + [pltpu.VMEM((2, K), jnp.int32)] * NI
          + [pltpu.VMEM((K, h), jnp.float32)] * NR
          + [pltpu.SemaphoreType.DMA] * (NI + 2 * NR),
    )
    def sc_kernel(table_hbm, idx_hbm, zeros_hbm, out_hbm, accum, *bufs):
        ibuf = bufs[:NI]
        rows = bufs[NI:NI + NR]
        sem_i = bufs[NI + NR:2 * NI + NR]
        sem_g = bufs[2 * NI + NR:2 * NI + 2 * NR]
        sem_s = bufs[2 * NI + 2 * NR:]
        cid = lax.axis_index("c")
        sid = lax.axis_index("s")
        r0 = sid * rpt
        pltpu.sync_copy(zeros_hbm.at[pl.ds(r0, rpt)], accum.at[pl.ds(r0, rpt)])
        if tail:
            @pl.when(sid == NS - 1)
            def _():
                pltpu.sync_copy(zeros_hbm.at[pl.ds(NS * rpt, tail)],
                                accum.at[pl.ds(NS * rpt, tail)])
        plsc.subcore_barrier()

        def idx_start(c, s):
            pltpu.async_copy(idx_hbm.at[c], ibuf[s], sem_i[s])

        def idx_wait(c, s):
            pltpu.make_async_copy(idx_hbm.at[c], ibuf[s], sem_i[s]).wait()

        def gather_start(s, rs):
            pltpu.async_copy(table_hbm.at[ibuf[s].at[0]], rows[rs], sem_g[rs])

        def gather_wait(s, rs):
            pltpu.make_async_copy(table_hbm.at[ibuf[s].at[0]], rows[rs],
                                  sem_g[rs]).wait()

        def scat_start(s, rs):
            pltpu.async_copy(rows[rs], accum.at[ibuf[s].at[1]], sem_s[rs],
                             add=True)

        def scat_wait(s, rs):
            pltpu.make_async_copy(rows[rs], accum.at[ibuf[s].at[1]],
                                  sem_s[rs]).wait()

        def run_pipeline(c0, tpw):
            for c in range(3):
                idx_start(c0 + c, c)
            for c in range(2):
                idx_wait(c0 + c, c)
                gather_start(c, c)

            @pl.loop(0, tpw // NU)
            def _(g):
                jg = g * NU
                for u in range(NU):
                    j = jg + u

                    @pl.when(j < tpw - 3)
                    def _(j=j, s=(u + 3) % NI):
                        idx_start(c0 + j + 3, s)

                    @pl.when(jnp.logical_and(j >= 2, j < tpw - 2))
                    def _(s=(u + NI - 2) % NI, rs=(u + NR - 2) % NR):
                        scat_wait(s, rs)

                    @pl.when(j < tpw - 2)
                    def _(j=j, s=(u + 2) % NI, rs=(u + 2) % NR):
                        idx_wait(c0 + j + 2, s)
                        gather_start(s, rs)

                    gather_wait(u % NI, u % NR)
                    scat_start(u % NI, u % NR)

            for c in range(tpw - NR, tpw):
                scat_wait(c % NI, c % NR)

        @pl.when(cid == 0)
        def _():
            run_pipeline(sid * tpw0, tpw0)

        @pl.when(cid == 1)
        def _():
            run_pipeline(NS * tpw0 + sid * tpw1, tpw1)

        plsc.subcore_barrier()
        pltpu.sync_copy(accum.at[pl.ds(r0, rpt)],
                        out_hbm.at[cid, pl.ds(r0, rpt)])
        if tail:
            @pl.when(sid == NS - 1)
            def _():
                pltpu.sync_copy(accum.at[pl.ds(NS * rpt, tail)],
                                out_hbm.at[cid, pl.ds(NS * rpt, tail)])

    return sc_kernel(table, idxpair, zeros)


def _tc_layer_call(xb, agg, wu, wd, w, blk):
    n, d = xb.shape
    h = wu.shape[1]
    nb = n // blk
    has_agg = agg is not None

    def body(*refs):
        if has_agg:
            xr, ar, wur, wdr, wr, t_ref, base_ref = refs
            x = xr[...] + ar[0] + ar[1]
        else:
            xr, wur, wdr, wr, t_ref, base_ref = refs
            x = xr[...]
        u = jnp.dot(x, wur[...], preferred_element_type=jnp.float32)
        v = jnp.dot(x, wdr[...], preferred_element_type=jnp.float32)
        t_ref[0] = u
        t_ref[1] = v
        t_ref[2] = -u
        t_ref[3] = -v
        t_ref[4] = jnp.zeros((blk, h), jnp.float32)
        base_ref[...] = jnp.dot(x, wr[...], preferred_element_type=jnp.float32)

    in_specs = [pl.BlockSpec((blk, d), lambda i: (i, 0))]
    args = [xb]
    if has_agg:
        in_specs.append(pl.BlockSpec((NC, blk, h), lambda i: (0, i, 0)))
        args.append(agg)
    in_specs += [pl.BlockSpec((d, h), lambda i: (0, 0))] * 3
    args += [wu, wd, w]

    return pl.pallas_call(
        body,
        grid=(nb,),
        in_specs=in_specs,
        out_specs=[
            pl.BlockSpec((5, blk, h), lambda i: (0, i, 0)),
            pl.BlockSpec((blk, h), lambda i: (i, 0)),
        ],
        out_shape=[
            jax.ShapeDtypeStruct((5, n, h), jnp.float32),
            jax.ShapeDtypeStruct((n, h), jnp.float32),
        ],
    )(*args)


def _tc_readout_call(base, agg, batch2d, w1, b1, w2, b2, nseg, blk):
    n, h = base.shape
    c = w2.shape[1]
    nb = n // blk

    def body(base_ref, a_ref, bt_ref, w1_ref, b1_ref, w2_ref, b2_ref,
             out_ref, pooled_ref):
        i = pl.program_id(0)
        x = base_ref[...] + a_ref[0] + a_ref[1]
        xa = jnp.abs(x)
        seg = bt_ref[...]
        onehot = (seg == lax.broadcasted_iota(jnp.int32, (blk, nseg), 1)
                  ).astype(jnp.float32)
        part = lax.dot_general(onehot, xa, (((0,), (0,)), ((), ())),
                               preferred_element_type=jnp.float32)

        @pl.when(i == 0)
        def _():
            pooled_ref[...] = part

        @pl.when(i > 0)
        def _():
            pooled_ref[...] += part

        @pl.when(i == nb - 1)
        def _():
            hdn = jnp.maximum(
                jnp.dot(pooled_ref[...], w1_ref[...],
                        preferred_element_type=jnp.float32) + b1_ref[...], 0.0)
            out_ref[...] = jnp.dot(hdn, w2_ref[...],
                                   preferred_element_type=jnp.float32) + b2_ref[...]

    return pl.pallas_call(
        body,
        grid=(nb,),
        in_specs=[
            pl.BlockSpec((blk, h), lambda i: (i, 0)),
            pl.BlockSpec((NC, blk, h), lambda i: (0, i, 0)),
            pl.BlockSpec((blk, 1), lambda i: (i, 0)),
            pl.BlockSpec((h, h), lambda i: (0, 0)),
            pl.BlockSpec((1, h), lambda i: (0, 0)),
            pl.BlockSpec((h, c), lambda i: (0, 0)),
            pl.BlockSpec((1, c), lambda i: (0, 0)),
        ],
        out_specs=pl.BlockSpec((nseg, c), lambda i: (0, 0)),
        out_shape=jax.ShapeDtypeStruct((nseg, c), jnp.float32),
        scratch_shapes=[pltpu.VMEM((nseg, h), jnp.float32)],
    )(base, agg, batch2d, w1, b1, w2, b2)


def kernel(x, up_index, up_orient, down_index, down_orient, batch,
           W_up_0, W_down_0, W_0, W_up_1, W_down_1, W_1,
           W_up_2, W_down_2, W_2, lin1_W, lin1_b, lin2_W, lin2_b):
    n, d = x.shape
    h = W_0.shape[1]
    e = up_index.shape[1]
    nseg = 64
    c = lin2_W.shape[1]
    blk = 1000

    up_g = jnp.where(up_orient > 0, up_index[0],
                     jnp.where(up_orient < 0, up_index[0] + 2 * n,
                               up_index[0] + 4 * n))
    dn_g = jnp.where(down_orient > 0, down_index[0] + n,
                     jnp.where(down_orient < 0, down_index[0] + 3 * n,
                               down_index[0] + 4 * n))
    gidx = jnp.concatenate([up_g, dn_g]).astype(jnp.int32)
    didx = jnp.concatenate([up_index[1], down_index[1]]).astype(jnp.int32)

    ep = 2 * e
    tch = -(-ep // (NS * K))
    tpw0 = max(NU, int(round(tch * FRAC0 / NU)) * NU)
    tpw1 = -(-(tch - tpw0) // NU) * NU
    epad = NS * (tpw0 + tpw1) * K
    pad = epad - ep
    if pad:
        gidx = jnp.concatenate([gidx, jnp.full((pad,), 4 * n, jnp.int32)])
        didx = jnp.concatenate(
            [didx, (jnp.arange(pad, dtype=jnp.int32) % n)])
    idxpair = jnp.stack([gidx.reshape(-1, K), didx.reshape(-1, K)], axis=1)

    zeros = jnp.zeros((n, h), jnp.float32)
    batch2d = batch.astype(jnp.int32).reshape(n, 1)

    wus = [W_up_0, W_up_1, W_up_2]
    wds = [W_down_0, W_down_1, W_down_2]
    ws = [W_0, W_1, W_2]

    xb, agg = x, None
    for l in range(3):
        t5, base = _tc_layer_call(xb, agg, wus[l], wds[l], ws[l], blk)
        agg = _sc_scatter_call(t5.reshape(5 * n, h), idxpair, zeros,
                               tpw0, tpw1)
        xb = base

    return _tc_readout_call(xb, agg, batch2d, lin1_W,
                            lin1_b.reshape(1, h), lin2_W,
                            lin2_b.reshape(1, c), nseg, blk)


# speedup vs baseline: 1.0395x; 1.0395x over previous
import functools

import jax
import jax.numpy as jnp
from jax import lax
from jax.experimental import pallas as pl
from jax.experimental.pallas import tpu as pltpu
from jax.experimental.pallas import tpu_sc as plsc

NC = 2
NS = 16
K = 88
NI = 8
NR = 4
NU = 8
FRAC0 = 0.68


def _sc_scatter_call(table, idxpair, zeros, tpw0, tpw1):
    n, h = zeros.shape
    rpt = (n // NS) & ~7
    tail = n - NS * rpt
    assert tpw0 % NU == 0 and tpw1 % NU == 0
    mesh = plsc.VectorSubcoreMesh(core_axis_name="c", subcore_axis_name="s")

    @functools.partial(
        pl.kernel,
        mesh=mesh,
        out_type=jax.ShapeDtypeStruct((NC, n, h), jnp.float32),
        scratch_types=[
            pltpu.VMEM_SHARED((n, h), jnp.float32),
        ] + [pltpu.VMEM((2, K), jnp.int32)] * NI
          + [pltpu.VMEM((K, h), jnp.float32)] * NR
          + [pltpu.SemaphoreType.DMA] * (NI + 2 * NR),
    )
    def sc_kernel(table_hbm, idx_hbm, zeros_hbm, out_hbm, accum, *bufs):
        ibuf = bufs[:NI]
        rows = bufs[NI:NI + NR]
        sem_i = bufs[NI + NR:2 * NI + NR]
        sem_g = bufs[2 * NI + NR:2 * NI + 2 * NR]
        sem_s = bufs[2 * NI + 2 * NR:]
        cid = lax.axis_index("c")
        sid = lax.axis_index("s")
        r0 = sid * rpt
        pltpu.sync_copy(zeros_hbm.at[pl.ds(r0, rpt)], accum.at[pl.ds(r0, rpt)])
        if tail:
            @pl.when(sid == NS - 1)
            def _():
                pltpu.sync_copy(zeros_hbm.at[pl.ds(NS * rpt, tail)],
                                accum.at[pl.ds(NS * rpt, tail)])
        plsc.subcore_barrier()

        def idx_start(c, s):
            pltpu.async_copy(idx_hbm.at[c], ibuf[s], sem_i[s])

        def idx_wait(c, s):
            pltpu.make_async_copy(idx_hbm.at[c], ibuf[s], sem_i[s]).wait()

        def gather_start(s, rs):
            pltpu.async_copy(table_hbm.at[ibuf[s].at[0]], rows[rs], sem_g[rs])

        def gather_wait(s, rs):
            pltpu.make_async_copy(table_hbm.at[ibuf[s].at[0]], rows[rs],
                                  sem_g[rs]).wait()

        def scat_start(s, rs):
            pltpu.async_copy(rows[rs], accum.at[ibuf[s].at[1]], sem_s[rs],
                             add=True)

        def scat_wait(s, rs):
            pltpu.make_async_copy(rows[rs], accum.at[ibuf[s].at[1]],
                                  sem_s[rs]).wait()

        def run_pipeline(c0, tpw):
            for c in range(3):
                idx_start(c0 + c, c)
            for c in range(2):
                idx_wait(c0 + c, c)
                gather_start(c, c)

            @pl.loop(0, tpw // NU)
            def _(g):
                jg = g * NU
                for u in range(NU):
                    j = jg + u

                    @pl.when(j < tpw - 3)
                    def _(j=j, s=(u + 3) % NI):
                        idx_start(c0 + j + 3, s)

                    @pl.when(jnp.logical_and(j >= 2, j < tpw - 2))
                    def _(s=(u + NI - 2) % NI, rs=(u + NR - 2) % NR):
                        scat_wait(s, rs)

                    @pl.when(j < tpw - 2)
                    def _(j=j, s=(u + 2) % NI, rs=(u + 2) % NR):
                        idx_wait(c0 + j + 2, s)
                        gather_start(s, rs)

                    gather_wait(u % NI, u % NR)
                    scat_start(u % NI, u % NR)

            for c in range(tpw - NR, tpw):
                scat_wait(c % NI, c % NR)

        @pl.when(cid == 0)
        def _():
            run_pipeline(sid * tpw0, tpw0)

        @pl.when(cid == 1)
        def _():
            run_pipeline(NS * tpw0 + sid * tpw1, tpw1)

        plsc.subcore_barrier()
        pltpu.sync_copy(accum.at[pl.ds(r0, rpt)],
                        out_hbm.at[cid, pl.ds(r0, rpt)])
        if tail:
            @pl.when(sid == NS - 1)
            def _():
                pltpu.sync_copy(accum.at[pl.ds(NS * rpt, tail)],
                                out_hbm.at[cid, pl.ds(NS * rpt, tail)])

    return sc_kernel(table, idxpair, zeros)


def _tc_layer_call(xb, agg, wu, wd, w, blk):
    n, d = xb.shape
    h = wu.shape[1]
    nb = n // blk
    has_agg = agg is not None

    def body(*refs):
        if has_agg:
            xr, ar, wur, wdr, wr, t_ref, base_ref = refs
            x = xr[...] + ar[0] + ar[1]
        else:
            xr, wur, wdr, wr, t_ref, base_ref = refs
            x = xr[...]
        u = jnp.dot(x, wur[...], preferred_element_type=jnp.float32)
        v = jnp.dot(x, wdr[...], preferred_element_type=jnp.float32)
        t_ref[0] = u
        t_ref[1] = v
        t_ref[2] = -u
        t_ref[3] = -v
        t_ref[4] = jnp.zeros((blk, h), jnp.float32)
        base_ref[...] = jnp.dot(x, wr[...], preferred_element_type=jnp.float32)

    in_specs = [pl.BlockSpec((blk, d), lambda i: (i, 0))]
    args = [xb]
    if has_agg:
        in_specs.append(pl.BlockSpec((NC, blk, h), lambda i: (0, i, 0)))
        args.append(agg)
    in_specs += [pl.BlockSpec((d, h), lambda i: (0, 0))] * 3
    args += [wu, wd, w]

    return pl.pallas_call(
        body,
        grid=(nb,),
        in_specs=in_specs,
        out_specs=[
            pl.BlockSpec((5, blk, h), lambda i: (0, i, 0)),
            pl.BlockSpec((blk, h), lambda i: (i, 0)),
        ],
        out_shape=[
            jax.ShapeDtypeStruct((5, n, h), jnp.float32),
            jax.ShapeDtypeStruct((n, h), jnp.float32),
        ],
    )(*args)


def _tc_readout_call(base, agg, batch2d, w1, b1, w2, b2, nseg, blk):
    n, h = base.shape
    c = w2.shape[1]
    nb = n // blk

    def body(base_ref, a_ref, bt_ref, w1_ref, b1_ref, w2_ref, b2_ref,
             out_ref, pooled_ref):
        i = pl.program_id(0)
        x = base_ref[...] + a_ref[0] + a_ref[1]
        xa = jnp.abs(x)
        seg = bt_ref[...]
        onehot = (seg == lax.broadcasted_iota(jnp.int32, (blk, nseg), 1)
                  ).astype(jnp.float32)
        part = lax.dot_general(onehot, xa, (((0,), (0,)), ((), ())),
                               preferred_element_type=jnp.float32)

        @pl.when(i == 0)
        def _():
            pooled_ref[...] = part

        @pl.when(i > 0)
        def _():
            pooled_ref[...] += part

        @pl.when(i == nb - 1)
        def _():
            hdn = jnp.maximum(
                jnp.dot(pooled_ref[...], w1_ref[...],
                        preferred_element_type=jnp.float32) + b1_ref[...], 0.0)
            out_ref[...] = jnp.dot(hdn, w2_ref[...],
                                   preferred_element_type=jnp.float32) + b2_ref[...]

    return pl.pallas_call(
        body,
        grid=(nb,),
        in_specs=[
            pl.BlockSpec((blk, h), lambda i: (i, 0)),
            pl.BlockSpec((NC, blk, h), lambda i: (0, i, 0)),
            pl.BlockSpec((blk, 1), lambda i: (i, 0)),
            pl.BlockSpec((h, h), lambda i: (0, 0)),
            pl.BlockSpec((1, h), lambda i: (0, 0)),
            pl.BlockSpec((h, c), lambda i: (0, 0)),
            pl.BlockSpec((1, c), lambda i: (0, 0)),
        ],
        out_specs=pl.BlockSpec((nseg, c), lambda i: (0, 0)),
        out_shape=jax.ShapeDtypeStruct((nseg, c), jnp.float32),
        scratch_shapes=[pltpu.VMEM((nseg, h), jnp.float32)],
    )(base, agg, batch2d, w1, b1, w2, b2)


def kernel(x, up_index, up_orient, down_index, down_orient, batch,
           W_up_0, W_down_0, W_0, W_up_1, W_down_1, W_1,
           W_up_2, W_down_2, W_2, lin1_W, lin1_b, lin2_W, lin2_b):
    n, d = x.shape
    h = W_0.shape[1]
    e = up_index.shape[1]
    nseg = 64
    c = lin2_W.shape[1]
    blk = 1000

    up_g = jnp.where(up_orient > 0, up_index[0],
                     jnp.where(up_orient < 0, up_index[0] + 2 * n,
                               up_index[0] + 4 * n))
    dn_g = jnp.where(down_orient > 0, down_index[0] + n,
                     jnp.where(down_orient < 0, down_index[0] + 3 * n,
                               down_index[0] + 4 * n))
    gidx = jnp.concatenate([up_g, dn_g]).astype(jnp.int32)
    didx = jnp.concatenate([up_index[1], down_index[1]]).astype(jnp.int32)

    ep = 2 * e
    tch = -(-ep // (NS * K))
    tpw0 = max(NU, int(round(tch * FRAC0 / NU)) * NU)
    tpw1 = -(-(tch - tpw0) // NU) * NU
    epad = NS * (tpw0 + tpw1) * K
    pad = epad - ep
    if pad:
        gidx = jnp.concatenate([gidx, jnp.full((pad,), 4 * n, jnp.int32)])
        didx = jnp.concatenate(
            [didx, (jnp.arange(pad, dtype=jnp.int32) % n)])
    idxpair = jnp.stack([gidx.reshape(-1, K), didx.reshape(-1, K)], axis=1)

    zeros = jnp.zeros((n, h), jnp.float32)
    batch2d = batch.astype(jnp.int32).reshape(n, 1)

    wus = [W_up_0, W_up_1, W_up_2]
    wds = [W_down_0, W_down_1, W_down_2]
    ws = [W_0, W_1, W_2]

    xb, agg = x, None
    for l in range(3):
        t5, base = _tc_layer_call(xb, agg, wus[l], wds[l], ws[l], blk)
        agg = _sc_scatter_call(t5.reshape(5 * n, h), idxpair, zeros,
                               tpw0, tpw1)
        xb = base

    return _tc_readout_call(xb, agg, batch2d, lin1_W,
                            lin1_b.reshape(1, h), lin2_W,
                            lin2_b.reshape(1, c), nseg, blk)
